# baseline (speedup 1.0000x reference)
import jax
import jax.numpy as jnp
from jax.experimental import pallas as pl
from jax.experimental.pallas import tpu as pltpu

T = 2048; D = 1024; HQ = 16; HKV = 8; HD = 64; E = 64; K = 2; F = 512; C = 128
EPS = 1e-6; THETA = 10000.0
BQ = 256
BR = 256
NEG = -1e30


def _bf(x):
    return x.astype(jnp.bfloat16)


def _rope(x, cos, sin):
    x1 = x[:, :HD // 2]
    x2 = x[:, HD // 2:]
    return jnp.concatenate([x1 * cos - x2 * sin, x2 * cos + x1 * sin], axis=1)


def _qkv_body(x_ref, g_ref, w_ref, cos_ref, sin_ref, q_ref, k_ref, v_ref):
    x = x_ref[...]
    var = jnp.mean(x * x, axis=1, keepdims=True)
    xn = x * jax.lax.rsqrt(var + EPS) * g_ref[...]
    o = jnp.dot(_bf(xn), _bf(w_ref[...]), preferred_element_type=jnp.float32)
    cos = cos_ref[...]
    sin = sin_ref[...]
    for h in range(HQ):
        q_ref[h] = _rope(o[:, h * HD:(h + 1) * HD], cos, sin)
    for h in range(HKV):
        k_ref[h] = _rope(o[:, (HQ + h) * HD:(HQ + h + 1) * HD], cos, sin)
        v_ref[h] = o[:, (HQ + HKV + h) * HD:(HQ + HKV + h + 1) * HD]


def _attn_body(q_ref, k_ref, v_ref, o_ref):
    qb = pl.program_id(1)
    dn = (((1,), (1,)), ((), ()))
    q16 = _bf(q_ref[0] * (HD ** -0.5))
    ones = jnp.ones((BQ, HD), jnp.bfloat16)

    def chunk(j, m, acc, masked):
        kj = _bf(k_ref[0, pl.ds(j * BQ, BQ), :])
        sj = jax.lax.dot_general(q16, kj, dn,
                                 preferred_element_type=jnp.float32)
        if masked:
            ri = jax.lax.broadcasted_iota(jnp.int32, (BQ, BQ), 0)
            ci = jax.lax.broadcasted_iota(jnp.int32, (BQ, BQ), 1)
            sj = jnp.where(ci <= ri, sj, NEG)
        mj = jnp.maximum(m, jnp.max(sj, axis=(0, 1), keepdims=True))
        pj = _bf(jnp.exp(sj - mj[0, 0]))
        vj = _bf(v_ref[0, pl.ds(j * BQ, BQ), :])
        v_ext = jnp.concatenate([vj, ones], axis=1)
        upd = jax.lax.dot_general(pj, v_ext, (((1,), (0,)), ((), ())),
                                  preferred_element_type=jnp.float32)
        corr = jnp.exp(m - mj)
        return mj, acc * corr[0, 0] + upd

    m0 = jnp.full((1, 1), NEG, jnp.float32)
    acc0 = jnp.zeros((BQ, 2 * HD), jnp.float32)
    m, acc = chunk(qb, m0, acc0, True)

    def body(j, carry):
        m, acc = carry
        return chunk(j, m, acc, False)

    m, acc = jax.lax.fori_loop(0, qb, body, (m, acc))
    o_ref[0] = acc[:, :HD] / acc[:, HD:HD + 1]


def _proj_body(o_ref, w_ref, res_ref, out_ref):
    o2d = jnp.concatenate([o_ref[h] for h in range(HQ)], axis=1)
    out_ref[...] = res_ref[...] + jnp.dot(
        _bf(o2d), _bf(w_ref[...]), preferred_element_type=jnp.float32)


def _route_body(hmid_ref, g_ref, gw_ref, h2_ref, tok_ref, gidx_ref, gwt_ref):
    h = hmid_ref[...]
    var = jnp.mean(h * h, axis=1, keepdims=True)
    h2 = h * jax.lax.rsqrt(var + EPS) * g_ref[...]
    h2_ref[...] = h2
    logits = jnp.dot(h2, gw_ref[...], preferred_element_type=jnp.float32)

    iota_e = jax.lax.broadcasted_iota(jnp.int32, (T, E), 1).astype(jnp.float32)
    m1 = jnp.max(logits, axis=1, keepdims=True)
    i1 = jnp.min(jnp.where(logits == m1, iota_e, float(E)), axis=1,
                 keepdims=True)
    o1 = (iota_e == i1).astype(jnp.float32)
    l2 = jnp.where(o1 > 0, NEG, logits)
    m2 = jnp.max(l2, axis=1, keepdims=True)
    i2 = jnp.min(jnp.where(l2 == m2, iota_e, float(E)), axis=1, keepdims=True)
    o2 = (iota_e == i2).astype(jnp.float32)
    e2 = jnp.exp(m2 - m1)
    wa = 1.0 / (1.0 + e2)
    wb = e2 / (1.0 + e2)

    S = o1 + o2
    tri = (jax.lax.broadcasted_iota(jnp.int32, (BR, BR), 0)
           > jax.lax.broadcasted_iota(jnp.int32, (BR, BR), 1)).astype(jnp.float32)
    parts = []
    base = jnp.zeros((1, E), jnp.float32)
    for b in range(T // BR):
        sb = S[b * BR:(b + 1) * BR]
        parts.append(jnp.dot(tri, sb, preferred_element_type=jnp.float32) + base)
        base = base + jnp.sum(sb, axis=0, keepdims=True)
    ex = jnp.concatenate(parts, axis=0)
    pos1 = jnp.sum(ex * o1, axis=1, keepdims=True)
    pos2 = jnp.sum(ex * o2, axis=1, keepdims=True)

    capped1 = jnp.minimum(pos1, float(C - 1))
    capped2 = jnp.minimum(pos2, float(C - 1))
    gidx_ref[...] = jnp.concatenate(
        [i1 * C + capped1, i2 * C + capped2], axis=1).astype(jnp.int32)
    gwt_ref[...] = jnp.concatenate(
        [wa * (pos1 < C), wb * (pos2 < C)], axis=1)

    iota_c = jax.lax.broadcasted_iota(jnp.int32, (T, C), 1).astype(jnp.float32)
    P1 = (iota_c == pos1).astype(jnp.float32)
    P2 = (iota_c == pos2).astype(jnp.float32)
    tf = jax.lax.broadcasted_iota(jnp.int32, (T, 1), 0).astype(jnp.float32)
    th = jnp.floor(tf / 16.0)
    tl = tf - th * 16.0
    dn = (((0,), (0,)), ((), ()))
    tokf = (jax.lax.dot_general(o1, P1 * th, dn, preferred_element_type=jnp.float32)
            + jax.lax.dot_general(o2, P2 * th, dn, preferred_element_type=jnp.float32)) * 16.0 \
        + (jax.lax.dot_general(o1, P1 * tl, dn, preferred_element_type=jnp.float32)
           + jax.lax.dot_general(o2, P2 * tl, dn, preferred_element_type=jnp.float32))
    tok_ref[...] = tokf.astype(jnp.int32)


def _expert_body(tok_ref, h2_ref, w1_ref, w3_ref, w2_ref, y_ref, xg):
    e = pl.program_id(0)

    def gather(c, carry):
        t = tok_ref[e, c]
        xg[pl.ds(c, 1), :] = h2_ref[pl.ds(t, 1), :]
        return carry

    jax.lax.fori_loop(0, C, gather, 0)
    x = _bf(xg[...])
    a = jnp.dot(x, _bf(w1_ref[0]), preferred_element_type=jnp.float32)
    b = jnp.dot(x, _bf(w3_ref[0]), preferred_element_type=jnp.float32)
    act = a * jax.nn.sigmoid(a) * b
    y_ref[0] = jnp.dot(_bf(act), _bf(w2_ref[0]),
                       preferred_element_type=jnp.float32)


def _comb_body(gidx_ref, gwt_ref, hmid_ref, y_ref, out_ref):
    pid = pl.program_id(0)

    def body(i, carry):
        t = pid * BR + i
        g1 = gidx_ref[2 * t]
        g2 = gidx_ref[2 * t + 1]
        w1 = gwt_ref[2 * t]
        w2 = gwt_ref[2 * t + 1]
        out_ref[pl.ds(i, 1), :] = (hmid_ref[pl.ds(i, 1), :]
                                   + w1 * y_ref[pl.ds(g1, 1), :]
                                   + w2 * y_ref[pl.ds(g2, 1), :])
        return carry

    jax.lax.fori_loop(0, BR, body, 0)


def kernel(hidden_states, positions, ln1_w, ln2_w, wqkv, wo, gate_w, w1, w3, w2):
    f32 = jnp.float32
    half = HD // 2
    inv_freq = 1.0 / (THETA ** (jnp.arange(half, dtype=f32) / half))
    ang = positions.astype(f32)[:, None] * inv_freq[None, :]
    cos = jnp.cos(ang)
    sin = jnp.sin(ang)

    q3, k3, v3 = pl.pallas_call(
        _qkv_body,
        grid=(T // BR,),
        in_specs=[
            pl.BlockSpec((BR, D), lambda i: (i, 0)),
            pl.BlockSpec((1, D), lambda i: (0, 0)),
            pl.BlockSpec((D, (HQ + 2 * HKV) * HD), lambda i: (0, 0)),
            pl.BlockSpec((BR, half), lambda i: (i, 0)),
            pl.BlockSpec((BR, half), lambda i: (i, 0)),
        ],
        out_specs=[
            pl.BlockSpec((HQ, BR, HD), lambda i: (0, i, 0)),
            pl.BlockSpec((HKV, BR, HD), lambda i: (0, i, 0)),
            pl.BlockSpec((HKV, BR, HD), lambda i: (0, i, 0)),
        ],
        out_shape=[
            jax.ShapeDtypeStruct((HQ, T, HD), f32),
            jax.ShapeDtypeStruct((HKV, T, HD), f32),
            jax.ShapeDtypeStruct((HKV, T, HD), f32),
        ],
    )(hidden_states, ln1_w.reshape(1, D), wqkv, cos, sin)

    attn = pl.pallas_call(
        _attn_body,
        grid=(HQ, T // BQ),
        in_specs=[
            pl.BlockSpec((1, BQ, HD), lambda h, qb: (h, qb, 0)),
            pl.BlockSpec((1, T, HD), lambda h, qb: (h // 2, 0, 0)),
            pl.BlockSpec((1, T, HD), lambda h, qb: (h // 2, 0, 0)),
        ],
        out_specs=pl.BlockSpec((1, BQ, HD), lambda h, qb: (h, qb, 0)),
        out_shape=jax.ShapeDtypeStruct((HQ, T, HD), f32),
    )(q3, k3, v3)

    hmid = pl.pallas_call(
        _proj_body,
        grid=(T // BR,),
        in_specs=[
            pl.BlockSpec((HQ, BR, HD), lambda i: (0, i, 0)),
            pl.BlockSpec((HQ * HD, D), lambda i: (0, 0)),
            pl.BlockSpec((BR, D), lambda i: (i, 0)),
        ],
        out_specs=pl.BlockSpec((BR, D), lambda i: (i, 0)),
        out_shape=jax.ShapeDtypeStruct((T, D), f32),
    )(attn, wo, hidden_states)

    h2, tok, gidx, gwt = pl.pallas_call(
        _route_body,
        grid=(1,),
        in_specs=[
            pl.BlockSpec((T, D), lambda i: (0, 0)),
            pl.BlockSpec((1, D), lambda i: (0, 0)),
            pl.BlockSpec((D, E), lambda i: (0, 0)),
        ],
        out_specs=[
            pl.BlockSpec((T, D), lambda i: (0, 0)),
            pl.BlockSpec((E, C), lambda i: (0, 0)),
            pl.BlockSpec((T, K), lambda i: (0, 0)),
            pl.BlockSpec((T, K), lambda i: (0, 0)),
        ],
        out_shape=[
            jax.ShapeDtypeStruct((T, D), f32),
            jax.ShapeDtypeStruct((E, C), jnp.int32),
            jax.ShapeDtypeStruct((T, K), jnp.int32),
            jax.ShapeDtypeStruct((T, K), f32),
        ],
    )(hmid, ln2_w.reshape(1, D), gate_w)

    y = pl.pallas_call(
        _expert_body,
        grid=(E,),
        in_specs=[
            pl.BlockSpec(memory_space=pltpu.SMEM),
            pl.BlockSpec((T, D), lambda e: (0, 0)),
            pl.BlockSpec((1, D, F), lambda e: (e, 0, 0)),
            pl.BlockSpec((1, D, F), lambda e: (e, 0, 0)),
            pl.BlockSpec((1, F, D), lambda e: (e, 0, 0)),
        ],
        out_specs=pl.BlockSpec((1, C, D), lambda e: (e, 0, 0)),
        out_shape=jax.ShapeDtypeStruct((E, C, D), f32),
        scratch_shapes=[pltpu.VMEM((C, D), f32)],
    )(tok, h2, w1, w3, w2)

    out = pl.pallas_call(
        _comb_body,
        grid=(T // BR,),
        in_specs=[
            pl.BlockSpec(memory_space=pltpu.SMEM),
            pl.BlockSpec(memory_space=pltpu.SMEM),
            pl.BlockSpec((BR, D), lambda i: (i, 0)),
            pl.BlockSpec((E * C, D), lambda i: (0, 0)),
        ],
        out_specs=pl.BlockSpec((BR, D), lambda i: (i, 0)),
        out_shape=jax.ShapeDtypeStruct((T, D), f32),
    )(gidx.reshape(-1), gwt.reshape(-1), hmid, y.reshape(E * C, D))

    return out


# speedup vs baseline: 1.2198x; 1.2198x over previous
import jax
import jax.numpy as jnp
from jax.experimental import pallas as pl
from jax.experimental.pallas import tpu as pltpu

T = 2048; D = 1024; HQ = 16; HKV = 8; HD = 64; E = 64; K = 2; F = 512; C = 128
EPS = 1e-6; THETA = 10000.0
BQ = 256
BR = 256
NEG = -1e30


def _bf(x):
    return x.astype(jnp.bfloat16)


def _rope(x, cos, sin):
    x1 = x[:, :HD // 2]
    x2 = x[:, HD // 2:]
    return jnp.concatenate([x1 * cos - x2 * sin, x2 * cos + x1 * sin], axis=1)


def _qkv_body(x_ref, g_ref, w_ref, cos_ref, sin_ref, q_ref, k_ref, v_ref):
    x = x_ref[...]
    var = jnp.mean(x * x, axis=1, keepdims=True)
    xn = x * jax.lax.rsqrt(var + EPS) * g_ref[...]
    o = jnp.dot(_bf(xn), _bf(w_ref[...]), preferred_element_type=jnp.float32)
    cos = cos_ref[...]
    sin = sin_ref[...]
    for h in range(HQ):
        q_ref[h] = _rope(o[:, h * HD:(h + 1) * HD], cos, sin)
    for h in range(HKV):
        k_ref[h] = _rope(o[:, (HQ + h) * HD:(HQ + h + 1) * HD], cos, sin)
        v_ref[h] = o[:, (HQ + HKV + h) * HD:(HQ + HKV + h + 1) * HD]


def _attn_body(q_ref, k_ref, v_ref, o_ref):
    qb = pl.program_id(1)
    dn = (((1,), (1,)), ((), ()))
    q16 = _bf(q_ref[0] * (HD ** -0.5))
    ones = jnp.ones((BQ, HD), jnp.bfloat16)

    def chunk(j, m, acc, masked):
        kj = _bf(k_ref[0, pl.ds(j * BQ, BQ), :])
        sj = jax.lax.dot_general(q16, kj, dn,
                                 preferred_element_type=jnp.float32)
        if masked:
            ri = jax.lax.broadcasted_iota(jnp.int32, (BQ, BQ), 0)
            ci = jax.lax.broadcasted_iota(jnp.int32, (BQ, BQ), 1)
            sj = jnp.where(ci <= ri, sj, NEG)
        mj = jnp.maximum(m, jnp.max(sj, axis=1, keepdims=True))
        pj = _bf(jnp.exp(sj - mj))
        vj = _bf(v_ref[0, pl.ds(j * BQ, BQ), :])
        v_ext = jnp.concatenate([vj, ones], axis=1)
        upd = jax.lax.dot_general(pj, v_ext, (((1,), (0,)), ((), ())),
                                  preferred_element_type=jnp.float32)
        corr = jnp.exp(m - mj)
        return mj, acc * corr + upd

    m0 = jnp.full((BQ, 1), NEG, jnp.float32)
    acc0 = jnp.zeros((BQ, 2 * HD), jnp.float32)
    m, acc = chunk(qb, m0, acc0, True)

    def body(j, carry):
        m, acc = carry
        return chunk(j, m, acc, False)

    m, acc = jax.lax.fori_loop(0, qb, body, (m, acc))
    o_ref[0] = acc[:, :HD] / acc[:, HD:HD + 1]


def _proj_body(o_ref, w_ref, res_ref, out_ref):
    o2d = jnp.concatenate([o_ref[h] for h in range(HQ)], axis=1)
    out_ref[...] = res_ref[...] + jnp.dot(
        _bf(o2d), _bf(w_ref[...]), preferred_element_type=jnp.float32)


def _route_body(hmid_ref, g_ref, gw_ref, h2_ref, tok_ref, gidx_ref, gwt_ref):
    h = hmid_ref[...]
    var = jnp.mean(h * h, axis=1, keepdims=True)
    h2 = h * jax.lax.rsqrt(var + EPS) * g_ref[...]
    h2_ref[...] = h2
    logits = jnp.dot(h2, gw_ref[...], preferred_element_type=jnp.float32)

    iota_e = jax.lax.broadcasted_iota(jnp.int32, (T, E), 1).astype(jnp.float32)
    m1 = jnp.max(logits, axis=1, keepdims=True)
    i1 = jnp.min(jnp.where(logits == m1, iota_e, float(E)), axis=1,
                 keepdims=True)
    o1 = (iota_e == i1).astype(jnp.float32)
    l2 = jnp.where(o1 > 0, NEG, logits)
    m2 = jnp.max(l2, axis=1, keepdims=True)
    i2 = jnp.min(jnp.where(l2 == m2, iota_e, float(E)), axis=1, keepdims=True)
    o2 = (iota_e == i2).astype(jnp.float32)
    e2 = jnp.exp(m2 - m1)
    wa = 1.0 / (1.0 + e2)
    wb = e2 / (1.0 + e2)

    S = o1 + o2
    tri = (jax.lax.broadcasted_iota(jnp.int32, (BR, BR), 0)
           > jax.lax.broadcasted_iota(jnp.int32, (BR, BR), 1)).astype(jnp.float32)
    parts = []
    base = jnp.zeros((1, E), jnp.float32)
    for b in range(T // BR):
        sb = S[b * BR:(b + 1) * BR]
        parts.append(jnp.dot(tri, sb, preferred_element_type=jnp.float32) + base)
        base = base + jnp.sum(sb, axis=0, keepdims=True)
    ex = jnp.concatenate(parts, axis=0)
    pos1 = jnp.sum(ex * o1, axis=1, keepdims=True)
    pos2 = jnp.sum(ex * o2, axis=1, keepdims=True)

    capped1 = jnp.minimum(pos1, float(C - 1))
    capped2 = jnp.minimum(pos2, float(C - 1))
    gidx_ref[...] = jnp.concatenate(
        [i1 * C + capped1, i2 * C + capped2], axis=1).astype(jnp.int32)
    gwt_ref[...] = jnp.concatenate(
        [wa * (pos1 < C), wb * (pos2 < C)], axis=1)

    iota_c = jax.lax.broadcasted_iota(jnp.int32, (T, C), 1).astype(jnp.float32)
    P1 = (iota_c == pos1).astype(jnp.float32)
    P2 = (iota_c == pos2).astype(jnp.float32)
    tf = jax.lax.broadcasted_iota(jnp.int32, (T, 1), 0).astype(jnp.float32)
    th = jnp.floor(tf / 16.0)
    tl = tf - th * 16.0
    dn = (((0,), (0,)), ((), ()))
    tokf = (jax.lax.dot_general(o1, P1 * th, dn, preferred_element_type=jnp.float32)
            + jax.lax.dot_general(o2, P2 * th, dn, preferred_element_type=jnp.float32)) * 16.0 \
        + (jax.lax.dot_general(o1, P1 * tl, dn, preferred_element_type=jnp.float32)
           + jax.lax.dot_general(o2, P2 * tl, dn, preferred_element_type=jnp.float32))
    tok_ref[...] = tokf.astype(jnp.int32)


def _expert_body(tok_ref, h2_ref, w1_ref, w3_ref, w2_ref, y_ref, xg):
    e = pl.program_id(0)

    for c in range(C):
        t = tok_ref[e, c]
        xg[pl.ds(c, 1), :] = h2_ref[pl.ds(t, 1), :]
    x = _bf(xg[...])
    a = jnp.dot(x, _bf(w1_ref[0]), preferred_element_type=jnp.float32)
    b = jnp.dot(x, _bf(w3_ref[0]), preferred_element_type=jnp.float32)
    act = a * jax.nn.sigmoid(a) * b
    y_ref[0] = jnp.dot(_bf(act), _bf(w2_ref[0]),
                       preferred_element_type=jnp.float32)


def _comb_body(gidx_ref, gwt_ref, hmid_ref, y_ref, out_ref):
    pid = pl.program_id(0)

    for i in range(BR):
        t = pid * BR + i
        g1 = gidx_ref[2 * t]
        g2 = gidx_ref[2 * t + 1]
        w1 = gwt_ref[2 * t]
        w2 = gwt_ref[2 * t + 1]
        out_ref[pl.ds(i, 1), :] = (hmid_ref[pl.ds(i, 1), :]
                                   + w1 * y_ref[pl.ds(g1, 1), :]
                                   + w2 * y_ref[pl.ds(g2, 1), :])


def kernel(hidden_states, positions, ln1_w, ln2_w, wqkv, wo, gate_w, w1, w3, w2):
    f32 = jnp.float32
    half = HD // 2
    inv_freq = 1.0 / (THETA ** (jnp.arange(half, dtype=f32) / half))
    ang = positions.astype(f32)[:, None] * inv_freq[None, :]
    cos = jnp.cos(ang)
    sin = jnp.sin(ang)

    q3, k3, v3 = pl.pallas_call(
        _qkv_body,
        grid=(T // BR,),
        in_specs=[
            pl.BlockSpec((BR, D), lambda i: (i, 0)),
            pl.BlockSpec((1, D), lambda i: (0, 0)),
            pl.BlockSpec((D, (HQ + 2 * HKV) * HD), lambda i: (0, 0)),
            pl.BlockSpec((BR, half), lambda i: (i, 0)),
            pl.BlockSpec((BR, half), lambda i: (i, 0)),
        ],
        out_specs=[
            pl.BlockSpec((HQ, BR, HD), lambda i: (0, i, 0)),
            pl.BlockSpec((HKV, BR, HD), lambda i: (0, i, 0)),
            pl.BlockSpec((HKV, BR, HD), lambda i: (0, i, 0)),
        ],
        out_shape=[
            jax.ShapeDtypeStruct((HQ, T, HD), f32),
            jax.ShapeDtypeStruct((HKV, T, HD), f32),
            jax.ShapeDtypeStruct((HKV, T, HD), f32),
        ],
    )(hidden_states, ln1_w.reshape(1, D), wqkv, cos, sin)

    attn = pl.pallas_call(
        _attn_body,
        grid=(HQ, T // BQ),
        in_specs=[
            pl.BlockSpec((1, BQ, HD), lambda h, qb: (h, qb, 0)),
            pl.BlockSpec((1, T, HD), lambda h, qb: (h // 2, 0, 0)),
            pl.BlockSpec((1, T, HD), lambda h, qb: (h // 2, 0, 0)),
        ],
        out_specs=pl.BlockSpec((1, BQ, HD), lambda h, qb: (h, qb, 0)),
        out_shape=jax.ShapeDtypeStruct((HQ, T, HD), f32),
    )(q3, k3, v3)

    hmid = pl.pallas_call(
        _proj_body,
        grid=(T // BR,),
        in_specs=[
            pl.BlockSpec((HQ, BR, HD), lambda i: (0, i, 0)),
            pl.BlockSpec((HQ * HD, D), lambda i: (0, 0)),
            pl.BlockSpec((BR, D), lambda i: (i, 0)),
        ],
        out_specs=pl.BlockSpec((BR, D), lambda i: (i, 0)),
        out_shape=jax.ShapeDtypeStruct((T, D), f32),
    )(attn, wo, hidden_states)

    h2, tok, gidx, gwt = pl.pallas_call(
        _route_body,
        grid=(1,),
        in_specs=[
            pl.BlockSpec((T, D), lambda i: (0, 0)),
            pl.BlockSpec((1, D), lambda i: (0, 0)),
            pl.BlockSpec((D, E), lambda i: (0, 0)),
        ],
        out_specs=[
            pl.BlockSpec((T, D), lambda i: (0, 0)),
            pl.BlockSpec((E, C), lambda i: (0, 0)),
            pl.BlockSpec((T, K), lambda i: (0, 0)),
            pl.BlockSpec((T, K), lambda i: (0, 0)),
        ],
        out_shape=[
            jax.ShapeDtypeStruct((T, D), f32),
            jax.ShapeDtypeStruct((E, C), jnp.int32),
            jax.ShapeDtypeStruct((T, K), jnp.int32),
            jax.ShapeDtypeStruct((T, K), f32),
        ],
    )(hmid, ln2_w.reshape(1, D), gate_w)

    y = pl.pallas_call(
        _expert_body,
        grid=(E,),
        in_specs=[
            pl.BlockSpec(memory_space=pltpu.SMEM),
            pl.BlockSpec((T, D), lambda e: (0, 0)),
            pl.BlockSpec((1, D, F), lambda e: (e, 0, 0)),
            pl.BlockSpec((1, D, F), lambda e: (e, 0, 0)),
            pl.BlockSpec((1, F, D), lambda e: (e, 0, 0)),
        ],
        out_specs=pl.BlockSpec((1, C, D), lambda e: (e, 0, 0)),
        out_shape=jax.ShapeDtypeStruct((E, C, D), f32),
        scratch_shapes=[pltpu.VMEM((C, D), f32)],
    )(tok, h2, w1, w3, w2)

    out = pl.pallas_call(
        _comb_body,
        grid=(T // BR,),
        in_specs=[
            pl.BlockSpec(memory_space=pltpu.SMEM),
            pl.BlockSpec(memory_space=pltpu.SMEM),
            pl.BlockSpec((BR, D), lambda i: (i, 0)),
            pl.BlockSpec((E * C, D), lambda i: (0, 0)),
        ],
        out_specs=pl.BlockSpec((BR, D), lambda i: (i, 0)),
        out_shape=jax.ShapeDtypeStruct((T, D), f32),
    )(gidx.reshape(-1), gwt.reshape(-1), hmid, y.reshape(E * C, D))

    return out


# speedup vs baseline: 1.8057x; 1.4803x over previous
import jax
import jax.numpy as jnp
from jax.experimental import pallas as pl
from jax.experimental.pallas import tpu as pltpu

T = 2048; D = 1024; HQ = 16; HKV = 8; HD = 64; E = 64; K = 2; F = 512; C = 128
EPS = 1e-6; THETA = 10000.0
BQ = 512
BR = 256
NEG = -1e30


def _bf(x):
    return x.astype(jnp.bfloat16)


def _rope(x, cos, sin):
    x1 = x[:, :HD // 2]
    x2 = x[:, HD // 2:]
    return jnp.concatenate([x1 * cos - x2 * sin, x2 * cos + x1 * sin], axis=1)


def _qkv_body(x_ref, g_ref, w_ref, cos_ref, sin_ref, q_ref, k_ref, v_ref):
    x = x_ref[...]
    var = jnp.mean(x * x, axis=1, keepdims=True)
    xn = x * jax.lax.rsqrt(var + EPS) * g_ref[...]
    o = jnp.dot(_bf(xn), _bf(w_ref[...]), preferred_element_type=jnp.float32)
    cos = cos_ref[...]
    sin = sin_ref[...]
    for h in range(HQ):
        q_ref[h] = _rope(o[:, h * HD:(h + 1) * HD], cos, sin)
    for h in range(HKV):
        k_ref[h] = _rope(o[:, (HQ + h) * HD:(HQ + h + 1) * HD], cos, sin)
        v_ref[h] = o[:, (HQ + HKV + h) * HD:(HQ + HKV + h + 1) * HD]


def _attn_body(q_ref, k_ref, v_ref, o_ref):
    qb = pl.program_id(1)
    dn = (((1,), (1,)), ((), ()))
    q16 = _bf(q_ref[0] * (HD ** -0.5))
    ones = jnp.ones((BQ, HD), jnp.bfloat16)

    def chunk(j, m, acc, masked):
        kj = _bf(k_ref[0, pl.ds(j * BQ, BQ), :])
        sj = jax.lax.dot_general(q16, kj, dn,
                                 preferred_element_type=jnp.float32)
        if masked:
            ri = jax.lax.broadcasted_iota(jnp.int32, (BQ, BQ), 0)
            ci = jax.lax.broadcasted_iota(jnp.int32, (BQ, BQ), 1)
            sj = jnp.where(ci <= ri, sj, NEG)
        mj = jnp.maximum(m, jnp.max(sj, axis=1, keepdims=True))
        pj = _bf(jnp.exp(sj - mj))
        vj = _bf(v_ref[0, pl.ds(j * BQ, BQ), :])
        v_ext = jnp.concatenate([vj, ones], axis=1)
        upd = jax.lax.dot_general(pj, v_ext, (((1,), (0,)), ((), ())),
                                  preferred_element_type=jnp.float32)
        corr = jnp.exp(m - mj)
        return mj, acc * corr + upd

    m0 = jnp.full((BQ, 1), NEG, jnp.float32)
    acc0 = jnp.zeros((BQ, 2 * HD), jnp.float32)
    m, acc = chunk(qb, m0, acc0, True)

    def body(j, carry):
        m, acc = carry
        return chunk(j, m, acc, False)

    m, acc = jax.lax.fori_loop(0, qb, body, (m, acc))
    o_ref[0] = acc[:, :HD] / acc[:, HD:HD + 1]


def _proj_body(o_ref, w_ref, res_ref, out_ref):
    o2d = jnp.concatenate([o_ref[h] for h in range(HQ)], axis=1)
    out_ref[...] = res_ref[...] + jnp.dot(
        _bf(o2d), _bf(w_ref[...]), preferred_element_type=jnp.float32)


def _route_body(hmid_ref, g_ref, gw_ref, h2_ref, tok_ref, gidx_ref, gwt_ref):
    h = hmid_ref[...]
    var = jnp.mean(h * h, axis=1, keepdims=True)
    h2 = h * jax.lax.rsqrt(var + EPS) * g_ref[...]
    h2_ref[...] = h2
    logits = jnp.dot(h2, gw_ref[...], preferred_element_type=jnp.float32)

    iota_e = jax.lax.broadcasted_iota(jnp.int32, (T, E), 1).astype(jnp.float32)
    m1 = jnp.max(logits, axis=1, keepdims=True)
    i1 = jnp.min(jnp.where(logits == m1, iota_e, float(E)), axis=1,
                 keepdims=True)
    o1 = (iota_e == i1).astype(jnp.float32)
    l2 = jnp.where(o1 > 0, NEG, logits)
    m2 = jnp.max(l2, axis=1, keepdims=True)
    i2 = jnp.min(jnp.where(l2 == m2, iota_e, float(E)), axis=1, keepdims=True)
    o2 = (iota_e == i2).astype(jnp.float32)
    e2 = jnp.exp(m2 - m1)
    wa = 1.0 / (1.0 + e2)
    wb = e2 / (1.0 + e2)

    S = o1 + o2
    tri = (jax.lax.broadcasted_iota(jnp.int32, (BR, BR), 0)
           > jax.lax.broadcasted_iota(jnp.int32, (BR, BR), 1)).astype(jnp.float32)
    parts = []
    base = jnp.zeros((1, E), jnp.float32)
    for b in range(T // BR):
        sb = S[b * BR:(b + 1) * BR]
        parts.append(jnp.dot(tri, sb, preferred_element_type=jnp.float32) + base)
        base = base + jnp.sum(sb, axis=0, keepdims=True)
    ex = jnp.concatenate(parts, axis=0)
    pos1 = jnp.sum(ex * o1, axis=1, keepdims=True)
    pos2 = jnp.sum(ex * o2, axis=1, keepdims=True)

    capped1 = jnp.minimum(pos1, float(C - 1))
    capped2 = jnp.minimum(pos2, float(C - 1))
    gidx_ref[...] = jnp.concatenate(
        [i1 * C + capped1, i2 * C + capped2], axis=1).astype(jnp.int32)
    gwt_ref[...] = jnp.concatenate(
        [wa * (pos1 < C), wb * (pos2 < C)], axis=1)

    iota_c = jax.lax.broadcasted_iota(jnp.int32, (T, C), 1).astype(jnp.float32)
    P1 = (iota_c == pos1).astype(jnp.float32)
    P2 = (iota_c == pos2).astype(jnp.float32)
    tf = jax.lax.broadcasted_iota(jnp.int32, (T, 1), 0).astype(jnp.float32)
    th = jnp.floor(tf / 16.0)
    tl = tf - th * 16.0
    dn = (((0,), (0,)), ((), ()))
    tokf = (jax.lax.dot_general(o1, P1 * th, dn, preferred_element_type=jnp.float32)
            + jax.lax.dot_general(o2, P2 * th, dn, preferred_element_type=jnp.float32)) * 16.0 \
        + (jax.lax.dot_general(o1, P1 * tl, dn, preferred_element_type=jnp.float32)
           + jax.lax.dot_general(o2, P2 * tl, dn, preferred_element_type=jnp.float32))
    tok_ref[...] = tokf.astype(jnp.int32)


def _expert_body(tok_ref, h2_ref, w1_ref, w3_ref, w2_ref, y_ref, xg):
    e = pl.program_id(0)

    for c in range(C):
        t = tok_ref[e, c]
        xg[pl.ds(c, 1), :] = h2_ref[pl.ds(t, 1), :]
    x = _bf(xg[...])
    a = jnp.dot(x, _bf(w1_ref[0]), preferred_element_type=jnp.float32)
    b = jnp.dot(x, _bf(w3_ref[0]), preferred_element_type=jnp.float32)
    act = a * jax.nn.sigmoid(a) * b
    y_ref[0] = jnp.dot(_bf(act), _bf(w2_ref[0]),
                       preferred_element_type=jnp.float32)


def _comb_body(gidx_ref, gwt_ref, hmid_ref, y_ref, out_ref):
    pid = pl.program_id(0)

    for i in range(BR):
        t = pid * BR + i
        g1 = gidx_ref[2 * t]
        g2 = gidx_ref[2 * t + 1]
        w1 = gwt_ref[2 * t]
        w2 = gwt_ref[2 * t + 1]
        out_ref[pl.ds(i, 1), :] = (hmid_ref[pl.ds(i, 1), :]
                                   + w1 * y_ref[pl.ds(g1, 1), :]
                                   + w2 * y_ref[pl.ds(g2, 1), :])


def kernel(hidden_states, positions, ln1_w, ln2_w, wqkv, wo, gate_w, w1, w3, w2):
    f32 = jnp.float32
    half = HD // 2
    inv_freq = 1.0 / (THETA ** (jnp.arange(half, dtype=f32) / half))
    ang = positions.astype(f32)[:, None] * inv_freq[None, :]
    cos = jnp.cos(ang)
    sin = jnp.sin(ang)

    q3, k3, v3 = pl.pallas_call(
        _qkv_body,
        grid=(T // BR,),
        in_specs=[
            pl.BlockSpec((BR, D), lambda i: (i, 0)),
            pl.BlockSpec((1, D), lambda i: (0, 0)),
            pl.BlockSpec((D, (HQ + 2 * HKV) * HD), lambda i: (0, 0)),
            pl.BlockSpec((BR, half), lambda i: (i, 0)),
            pl.BlockSpec((BR, half), lambda i: (i, 0)),
        ],
        out_specs=[
            pl.BlockSpec((HQ, BR, HD), lambda i: (0, i, 0)),
            pl.BlockSpec((HKV, BR, HD), lambda i: (0, i, 0)),
            pl.BlockSpec((HKV, BR, HD), lambda i: (0, i, 0)),
        ],
        out_shape=[
            jax.ShapeDtypeStruct((HQ, T, HD), f32),
            jax.ShapeDtypeStruct((HKV, T, HD), f32),
            jax.ShapeDtypeStruct((HKV, T, HD), f32),
        ],
    )(hidden_states, ln1_w.reshape(1, D), wqkv, cos, sin)

    attn = pl.pallas_call(
        _attn_body,
        grid=(HQ, T // BQ),
        in_specs=[
            pl.BlockSpec((1, BQ, HD), lambda h, qb: (h, qb, 0)),
            pl.BlockSpec((1, T, HD), lambda h, qb: (h // 2, 0, 0)),
            pl.BlockSpec((1, T, HD), lambda h, qb: (h // 2, 0, 0)),
        ],
        out_specs=pl.BlockSpec((1, BQ, HD), lambda h, qb: (h, qb, 0)),
        out_shape=jax.ShapeDtypeStruct((HQ, T, HD), f32),
    )(q3, k3, v3)

    hmid = pl.pallas_call(
        _proj_body,
        grid=(T // BR,),
        in_specs=[
            pl.BlockSpec((HQ, BR, HD), lambda i: (0, i, 0)),
            pl.BlockSpec((HQ * HD, D), lambda i: (0, 0)),
            pl.BlockSpec((BR, D), lambda i: (i, 0)),
        ],
        out_specs=pl.BlockSpec((BR, D), lambda i: (i, 0)),
        out_shape=jax.ShapeDtypeStruct((T, D), f32),
    )(attn, wo, hidden_states)

    h2, tok, gidx, gwt = pl.pallas_call(
        _route_body,
        grid=(1,),
        in_specs=[
            pl.BlockSpec((T, D), lambda i: (0, 0)),
            pl.BlockSpec((1, D), lambda i: (0, 0)),
            pl.BlockSpec((D, E), lambda i: (0, 0)),
        ],
        out_specs=[
            pl.BlockSpec((T, D), lambda i: (0, 0)),
            pl.BlockSpec((E, C), lambda i: (0, 0)),
            pl.BlockSpec((T, K), lambda i: (0, 0)),
            pl.BlockSpec((T, K), lambda i: (0, 0)),
        ],
        out_shape=[
            jax.ShapeDtypeStruct((T, D), f32),
            jax.ShapeDtypeStruct((E, C), jnp.int32),
            jax.ShapeDtypeStruct((T, K), jnp.int32),
            jax.ShapeDtypeStruct((T, K), f32),
        ],
    )(hmid, ln2_w.reshape(1, D), gate_w)

    y = pl.pallas_call(
        _expert_body,
        grid=(E,),
        in_specs=[
            pl.BlockSpec(memory_space=pltpu.SMEM),
            pl.BlockSpec((T, D), lambda e: (0, 0)),
            pl.BlockSpec((1, D, F), lambda e: (e, 0, 0)),
            pl.BlockSpec((1, D, F), lambda e: (e, 0, 0)),
            pl.BlockSpec((1, F, D), lambda e: (e, 0, 0)),
        ],
        out_specs=pl.BlockSpec((1, C, D), lambda e: (e, 0, 0)),
        out_shape=jax.ShapeDtypeStruct((E, C, D), f32),
        scratch_shapes=[pltpu.VMEM((C, D), f32)],
    )(tok, h2, w1, w3, w2)

    out = pl.pallas_call(
        _comb_body,
        grid=(T // BR,),
        in_specs=[
            pl.BlockSpec(memory_space=pltpu.SMEM),
            pl.BlockSpec(memory_space=pltpu.SMEM),
            pl.BlockSpec((BR, D), lambda i: (i, 0)),
            pl.BlockSpec((E * C, D), lambda i: (0, 0)),
        ],
        out_specs=pl.BlockSpec((BR, D), lambda i: (i, 0)),
        out_shape=jax.ShapeDtypeStruct((T, D), f32),
    )(gidx.reshape(-1), gwt.reshape(-1), hmid, y.reshape(E * C, D))

    return out


# speedup vs baseline: 2.0244x; 1.1211x over previous
import jax
import jax.numpy as jnp
from jax.experimental import pallas as pl
from jax.experimental.pallas import tpu as pltpu

T = 2048; D = 1024; HQ = 16; HKV = 8; HD = 64; E = 64; K = 2; F = 512; C = 128
EPS = 1e-6; THETA = 10000.0
BQ = 1024
BR = 256
NEG = -1e30


def _bf(x):
    return x.astype(jnp.bfloat16)


def _rope(x, cos, sin):
    x1 = x[:, :HD // 2]
    x2 = x[:, HD // 2:]
    return jnp.concatenate([x1 * cos - x2 * sin, x2 * cos + x1 * sin], axis=1)


def _qkv_body(x_ref, g_ref, w_ref, cos_ref, sin_ref, q_ref, k_ref, v_ref):
    x = x_ref[...]
    var = jnp.mean(x * x, axis=1, keepdims=True)
    xn = x * jax.lax.rsqrt(var + EPS) * g_ref[...]
    o = jnp.dot(_bf(xn), _bf(w_ref[...]), preferred_element_type=jnp.float32)
    cos = cos_ref[...]
    sin = sin_ref[...]
    for h in range(HQ):
        q_ref[h] = _rope(o[:, h * HD:(h + 1) * HD], cos, sin)
    for h in range(HKV):
        k_ref[h] = _rope(o[:, (HQ + h) * HD:(HQ + h + 1) * HD], cos, sin)
        v_ref[h] = o[:, (HQ + HKV + h) * HD:(HQ + HKV + h + 1) * HD]


def _attn_body(q_ref, k_ref, v_ref, o_ref):
    qb = pl.program_id(1)
    dn = (((1,), (1,)), ((), ()))
    q16 = _bf(q_ref[0] * (HD ** -0.5))
    ones = jnp.ones((BQ, HD), jnp.bfloat16)

    def chunk(j, m, acc, masked):
        kj = _bf(k_ref[0, pl.ds(j * BQ, BQ), :])
        sj = jax.lax.dot_general(q16, kj, dn,
                                 preferred_element_type=jnp.float32)
        if masked:
            ri = jax.lax.broadcasted_iota(jnp.int32, (BQ, BQ), 0)
            ci = jax.lax.broadcasted_iota(jnp.int32, (BQ, BQ), 1)
            sj = jnp.where(ci <= ri, sj, NEG)
        mj = jnp.maximum(m, jnp.max(sj, axis=1, keepdims=True))
        pj = _bf(jnp.exp(sj - mj))
        vj = _bf(v_ref[0, pl.ds(j * BQ, BQ), :])
        v_ext = jnp.concatenate([vj, ones], axis=1)
        upd = jax.lax.dot_general(pj, v_ext, (((1,), (0,)), ((), ())),
                                  preferred_element_type=jnp.float32)
        corr = jnp.exp(m - mj)
        return mj, acc * corr + upd

    m0 = jnp.full((BQ, 1), NEG, jnp.float32)
    acc0 = jnp.zeros((BQ, 2 * HD), jnp.float32)
    m, acc = chunk(qb, m0, acc0, True)

    def body(j, carry):
        m, acc = carry
        return chunk(j, m, acc, False)

    m, acc = jax.lax.fori_loop(0, qb, body, (m, acc))
    o_ref[0] = acc[:, :HD] / acc[:, HD:HD + 1]


def _proj_body(o_ref, w_ref, res_ref, out_ref):
    o2d = jnp.concatenate([o_ref[h] for h in range(HQ)], axis=1)
    out_ref[...] = res_ref[...] + jnp.dot(
        _bf(o2d), _bf(w_ref[...]), preferred_element_type=jnp.float32)


def _route_body(hmid_ref, g_ref, gw_ref, h2_ref, tok_ref, gidx_ref, gwt_ref):
    h = hmid_ref[...]
    var = jnp.mean(h * h, axis=1, keepdims=True)
    h2 = h * jax.lax.rsqrt(var + EPS) * g_ref[...]
    h2_ref[...] = h2
    logits = jnp.dot(h2, gw_ref[...], preferred_element_type=jnp.float32)

    iota_e = jax.lax.broadcasted_iota(jnp.int32, (T, E), 1).astype(jnp.float32)
    m1 = jnp.max(logits, axis=1, keepdims=True)
    i1 = jnp.min(jnp.where(logits == m1, iota_e, float(E)), axis=1,
                 keepdims=True)
    o1 = (iota_e == i1).astype(jnp.float32)
    l2 = jnp.where(o1 > 0, NEG, logits)
    m2 = jnp.max(l2, axis=1, keepdims=True)
    i2 = jnp.min(jnp.where(l2 == m2, iota_e, float(E)), axis=1, keepdims=True)
    o2 = (iota_e == i2).astype(jnp.float32)
    e2 = jnp.exp(m2 - m1)
    wa = 1.0 / (1.0 + e2)
    wb = e2 / (1.0 + e2)

    S = o1 + o2
    tri = (jax.lax.broadcasted_iota(jnp.int32, (BR, BR), 0)
           > jax.lax.broadcasted_iota(jnp.int32, (BR, BR), 1)).astype(jnp.float32)
    parts = []
    base = jnp.zeros((1, E), jnp.float32)
    for b in range(T // BR):
        sb = S[b * BR:(b + 1) * BR]
        parts.append(jnp.dot(tri, sb, preferred_element_type=jnp.float32) + base)
        base = base + jnp.sum(sb, axis=0, keepdims=True)
    ex = jnp.concatenate(parts, axis=0)
    pos1 = jnp.sum(ex * o1, axis=1, keepdims=True)
    pos2 = jnp.sum(ex * o2, axis=1, keepdims=True)

    capped1 = jnp.minimum(pos1, float(C - 1))
    capped2 = jnp.minimum(pos2, float(C - 1))
    gidx_ref[...] = jnp.concatenate(
        [i1 * C + capped1, i2 * C + capped2], axis=1).astype(jnp.int32)
    gwt_ref[...] = jnp.concatenate(
        [wa * (pos1 < C), wb * (pos2 < C)], axis=1)

    iota_c = jax.lax.broadcasted_iota(jnp.int32, (T, C), 1).astype(jnp.float32)
    P1 = (iota_c == pos1).astype(jnp.float32)
    P2 = (iota_c == pos2).astype(jnp.float32)
    tf = jax.lax.broadcasted_iota(jnp.int32, (T, 1), 0).astype(jnp.float32)
    th = jnp.floor(tf / 16.0)
    tl = tf - th * 16.0
    dn = (((0,), (0,)), ((), ()))
    tokf = (jax.lax.dot_general(o1, P1 * th, dn, preferred_element_type=jnp.float32)
            + jax.lax.dot_general(o2, P2 * th, dn, preferred_element_type=jnp.float32)) * 16.0 \
        + (jax.lax.dot_general(o1, P1 * tl, dn, preferred_element_type=jnp.float32)
           + jax.lax.dot_general(o2, P2 * tl, dn, preferred_element_type=jnp.float32))
    tok_ref[...] = tokf.astype(jnp.int32)


def _expert_body(tok_ref, h2_ref, w1_ref, w3_ref, w2_ref, y_ref, xg):
    e = pl.program_id(0)

    for c in range(C):
        t = tok_ref[e, c]
        xg[pl.ds(c, 1), :] = h2_ref[pl.ds(t, 1), :]
    x = _bf(xg[...])
    a = jnp.dot(x, _bf(w1_ref[0]), preferred_element_type=jnp.float32)
    b = jnp.dot(x, _bf(w3_ref[0]), preferred_element_type=jnp.float32)
    act = a * jax.nn.sigmoid(a) * b
    y_ref[0] = jnp.dot(_bf(act), _bf(w2_ref[0]),
                       preferred_element_type=jnp.float32)


def _comb_body(gidx_ref, gwt_ref, hmid_ref, y_ref, out_ref):
    pid = pl.program_id(0)

    for i in range(BR):
        t = pid * BR + i
        g1 = gidx_ref[2 * t]
        g2 = gidx_ref[2 * t + 1]
        w1 = gwt_ref[2 * t]
        w2 = gwt_ref[2 * t + 1]
        out_ref[pl.ds(i, 1), :] = (hmid_ref[pl.ds(i, 1), :]
                                   + w1 * y_ref[pl.ds(g1, 1), :]
                                   + w2 * y_ref[pl.ds(g2, 1), :])


def kernel(hidden_states, positions, ln1_w, ln2_w, wqkv, wo, gate_w, w1, w3, w2):
    f32 = jnp.float32
    half = HD // 2
    inv_freq = 1.0 / (THETA ** (jnp.arange(half, dtype=f32) / half))
    ang = positions.astype(f32)[:, None] * inv_freq[None, :]
    cos = jnp.cos(ang)
    sin = jnp.sin(ang)

    q3, k3, v3 = pl.pallas_call(
        _qkv_body,
        grid=(T // BR,),
        in_specs=[
            pl.BlockSpec((BR, D), lambda i: (i, 0)),
            pl.BlockSpec((1, D), lambda i: (0, 0)),
            pl.BlockSpec((D, (HQ + 2 * HKV) * HD), lambda i: (0, 0)),
            pl.BlockSpec((BR, half), lambda i: (i, 0)),
            pl.BlockSpec((BR, half), lambda i: (i, 0)),
        ],
        out_specs=[
            pl.BlockSpec((HQ, BR, HD), lambda i: (0, i, 0)),
            pl.BlockSpec((HKV, BR, HD), lambda i: (0, i, 0)),
            pl.BlockSpec((HKV, BR, HD), lambda i: (0, i, 0)),
        ],
        out_shape=[
            jax.ShapeDtypeStruct((HQ, T, HD), f32),
            jax.ShapeDtypeStruct((HKV, T, HD), f32),
            jax.ShapeDtypeStruct((HKV, T, HD), f32),
        ],
    )(hidden_states, ln1_w.reshape(1, D), wqkv, cos, sin)

    attn = pl.pallas_call(
        _attn_body,
        grid=(HQ, T // BQ),
        in_specs=[
            pl.BlockSpec((1, BQ, HD), lambda h, qb: (h, qb, 0)),
            pl.BlockSpec((1, T, HD), lambda h, qb: (h // 2, 0, 0)),
            pl.BlockSpec((1, T, HD), lambda h, qb: (h // 2, 0, 0)),
        ],
        out_specs=pl.BlockSpec((1, BQ, HD), lambda h, qb: (h, qb, 0)),
        out_shape=jax.ShapeDtypeStruct((HQ, T, HD), f32),
    )(q3, k3, v3)

    hmid = pl.pallas_call(
        _proj_body,
        grid=(T // BR,),
        in_specs=[
            pl.BlockSpec((HQ, BR, HD), lambda i: (0, i, 0)),
            pl.BlockSpec((HQ * HD, D), lambda i: (0, 0)),
            pl.BlockSpec((BR, D), lambda i: (i, 0)),
        ],
        out_specs=pl.BlockSpec((BR, D), lambda i: (i, 0)),
        out_shape=jax.ShapeDtypeStruct((T, D), f32),
    )(attn, wo, hidden_states)

    h2, tok, gidx, gwt = pl.pallas_call(
        _route_body,
        grid=(1,),
        in_specs=[
            pl.BlockSpec((T, D), lambda i: (0, 0)),
            pl.BlockSpec((1, D), lambda i: (0, 0)),
            pl.BlockSpec((D, E), lambda i: (0, 0)),
        ],
        out_specs=[
            pl.BlockSpec((T, D), lambda i: (0, 0)),
            pl.BlockSpec((E, C), lambda i: (0, 0)),
            pl.BlockSpec((T, K), lambda i: (0, 0)),
            pl.BlockSpec((T, K), lambda i: (0, 0)),
        ],
        out_shape=[
            jax.ShapeDtypeStruct((T, D), f32),
            jax.ShapeDtypeStruct((E, C), jnp.int32),
            jax.ShapeDtypeStruct((T, K), jnp.int32),
            jax.ShapeDtypeStruct((T, K), f32),
        ],
    )(hmid, ln2_w.reshape(1, D), gate_w)

    y = pl.pallas_call(
        _expert_body,
        grid=(E,),
        in_specs=[
            pl.BlockSpec(memory_space=pltpu.SMEM),
            pl.BlockSpec((T, D), lambda e: (0, 0)),
            pl.BlockSpec((1, D, F), lambda e: (e, 0, 0)),
            pl.BlockSpec((1, D, F), lambda e: (e, 0, 0)),
            pl.BlockSpec((1, F, D), lambda e: (e, 0, 0)),
        ],
        out_specs=pl.BlockSpec((1, C, D), lambda e: (e, 0, 0)),
        out_shape=jax.ShapeDtypeStruct((E, C, D), f32),
        scratch_shapes=[pltpu.VMEM((C, D), f32)],
    )(tok, h2, w1, w3, w2)

    out = pl.pallas_call(
        _comb_body,
        grid=(T // BR,),
        in_specs=[
            pl.BlockSpec(memory_space=pltpu.SMEM),
            pl.BlockSpec(memory_space=pltpu.SMEM),
            pl.BlockSpec((BR, D), lambda i: (i, 0)),
            pl.BlockSpec((E * C, D), lambda i: (0, 0)),
        ],
        out_specs=pl.BlockSpec((BR, D), lambda i: (i, 0)),
        out_shape=jax.ShapeDtypeStruct((T, D), f32),
    )(gidx.reshape(-1), gwt.reshape(-1), hmid, y.reshape(E * C, D))

    return out
